# baseline (speedup 1.0000x reference)
import functools

import jax
import jax.numpy as jnp
from jax import lax
from jax.experimental import pallas as pl
from jax.experimental.pallas import tpu as pltpu
from jax.experimental.pallas import tpu_sc as plsc

_B, _S = 4096, 50
_D = 32
_NC, _NS = 2, 16
_NW = _NC * _NS
_BB = _B // _NW


_NFULL = 7812
_TAIL = 1000000 - _NFULL * 128


@jax.jit
def _relayout(tt, tail):
    mesh = plsc.VectorSubcoreMesh(core_axis_name="c", subcore_axis_name="s")

    @functools.partial(
        pl.kernel,
        mesh=mesh,
        out_type=jax.ShapeDtypeStruct((250000, 128), jnp.float32),
        scratch_types=(
            [pltpu.VMEM((_D, 128), jnp.float32) for _ in range(8)]
            + [pltpu.VMEM((16, 128), jnp.float32)]
            + [pltpu.SemaphoreType.DMA for _ in range(8)]
        ),
        compiler_params=pltpu.CompilerParams(
            use_tc_tiling_on_sc=True, needs_layout_passes=False
        ),
    )
    def k(tt_hbm, tail_hbm, out, i0, i1, i2, i3, o0, o1, o2, o3, tb,
          gs0, gs1, gs2, gs3, ss0, ss1, ss2, ss3):
        w = lax.axis_index("s") * _NC + lax.axis_index("c")
        ibuf, obuf = (i0, i1, i2, i3), (o0, o1, o2, o3)
        gsem, ssem = (gs0, gs1, gs2, gs3), (ss0, ss1, ss2, ss3)
        lanes = lax.iota(jnp.int32, 16)
        rowvec = (lanes, lanes + 16)
        nb = 244 + jnp.where(w < 4, 1, 0)

        def vblk(kk):
            return w + _NW * kk

        def fire(kk, par):
            pltpu.async_copy(
                tt_hbm.at[:, pl.ds(vblk(kk) * 128, 128)], ibuf[par], gsem[par]
            )

        def drain_gather(par):
            pltpu.make_async_copy(
                tt_hbm.at[:, pl.ds(0, 128)], ibuf[par], gsem[par]
            ).wait()

        def drain_store(par):
            pltpu.make_async_copy(
                tt_hbm.at[:, pl.ds(0, 128)], obuf[par], ssem[par]
            ).wait()

        def do_transpose(par):
            g, t = ibuf[par], obuf[par]
            for r in range(0, _D, 2):
                vals = []
                for m in range(16):
                    rr, p = r + m // 8, m % 8
                    col = jnp.full((16,), 4 * rr + p // 2, jnp.int32)
                    vals.append(plsc.load_gather(g, [rowvec[p % 2], col]))
                for m in range(16):
                    rr, p = r + m // 8, m % 8
                    t[rr, pl.ds(16 * p, 16)] = vals[m]

        def handle(i, kk, par):
            @pl.when(kk < nb)
            def _():
                drain_gather(par)

                @pl.when(kk + 3 < nb)
                def _():
                    fire(kk + 3, (par + 3) % 4)

                @pl.when(i > 0)
                def _():
                    drain_store(par)

                do_transpose(par)
                pltpu.async_copy(
                    obuf[par], out.at[pl.ds(vblk(kk) * _D, _D)], ssem[par]
                )

        fire(0, 0)
        fire(1, 1)
        fire(2, 2)

        def body(i, carry):
            for par in range(4):
                handle(i, 4 * i + par, par)
            return carry

        lax.fori_loop(0, 62, body, 0)
        for par in range(4):
            drain_store(par)

        @pl.when(w == 0)
        def _():
            pltpu.sync_copy(tail_hbm, tb)
            pltpu.sync_copy(tb, out.at[pl.ds(_NFULL * _D, 16)])

    return k(tt, tail)


@jax.jit
def _embed(t128, xt):
    mesh = plsc.VectorSubcoreMesh(core_axis_name="c", subcore_axis_name="s")

    @functools.partial(
        pl.kernel,
        mesh=mesh,
        out_type=jax.ShapeDtypeStruct((_S, _D, _B), jnp.float32),
        scratch_types=[
            pltpu.VMEM((_S, _BB), jnp.int32),
            pltpu.VMEM((_S, _BB), jnp.int32),
            pltpu.VMEM((_BB, 128), jnp.float32),
            pltpu.VMEM((_BB, 128), jnp.float32),
            pltpu.VMEM((_D, _BB), jnp.float32),
            pltpu.VMEM((_D, _BB), jnp.float32),
            pltpu.SemaphoreType.DMA,
            pltpu.SemaphoreType.DMA,
            pltpu.SemaphoreType.DMA,
            pltpu.SemaphoreType.DMA,
        ],
        compiler_params=pltpu.CompilerParams(
            use_tc_tiling_on_sc=True, needs_layout_passes=False
        ),
    )
    def k(tbl, xt_hbm, out, idx_v, jv, g0, g1, t0, t1, gs0, gs1, ss0, ss1):
        w = lax.axis_index("s") * _NC + lax.axis_index("c")
        pltpu.sync_copy(xt_hbm.at[:, pl.ds(w * _BB, _BB)], idx_v)

        gbuf, tbuf = (g0, g1), (t0, t1)
        gsem, ssem = (gs0, gs1), (ss0, ss1)
        lanes = lax.iota(jnp.int32, 16)
        rowvec = [lanes + 16 * p for p in range(8)]

        def prep(s, carry):
            for p in range(8):
                iv = idx_v[s, pl.ds(16 * p, 16)]
                jv[s, pl.ds(16 * p, 16)] = lax.shift_right_logical(iv, 2)
            return carry

        lax.fori_loop(0, _S, prep, 0)

        def fire(s, par):
            pltpu.async_copy(tbl.at[jv.at[s]], gbuf[par], gsem[par])

        def drain_gather(par):
            pltpu.make_async_copy(
                tbl.at[pl.ds(0, _BB)], gbuf[par], gsem[par]
            ).wait()

        def drain_store(par):
            pltpu.make_async_copy(
                out.at[0, :, pl.ds(0, _BB)], tbuf[par], ssem[par]
            ).wait()

        def transpose_select(s, par):
            g, t = gbuf[par], tbuf[par]
            q32 = []
            for p in range(8):
                iv = idx_v[s, pl.ds(16 * p, 16)]
                q32.append(lax.shift_left(jnp.bitwise_and(iv, 3), 5))
            for m0 in range(0, 2 * _BB, 8):
                vals = []
                for m in range(m0, m0 + 8):
                    d, p = m // 8, m % 8
                    vals.append(plsc.load_gather(g, [rowvec[p], q32[p] + d]))
                for m in range(m0, m0 + 8):
                    d, p = m // 8, m % 8
                    t[d, pl.ds(16 * p, 16)] = vals[m - m0]

        def handle(i, s, par):
            drain_gather(par)

            @pl.when(s + 1 < _S)
            def _():
                fire(s + 1, 1 - par)

            @pl.when(i > 0)
            def _():
                drain_store(par)

            transpose_select(s, par)
            pltpu.async_copy(
                tbuf[par], out.at[s, :, pl.ds(w * _BB, _BB)], ssem[par]
            )

        fire(0, 0)

        def body(i, carry):
            handle(i, 2 * i, 0)
            handle(i, 2 * i + 1, 1)
            return carry

        lax.fori_loop(0, _S // 2, body, 0)
        drain_store(0)
        drain_store(1)

    return k(t128, xt)


def kernel(x, table):
    tt = jnp.transpose(table)
    tail = table[_NFULL * 128:].reshape(16, 128)
    t128 = _relayout(tt, tail)
    xt = jnp.transpose(x)
    out = _embed(t128, xt)
    return jnp.transpose(out, (2, 0, 1))


# speedup vs baseline: 1.0530x; 1.0530x over previous
import functools

import jax
import jax.numpy as jnp
from jax import lax
from jax.experimental import pallas as pl
from jax.experimental.pallas import tpu as pltpu
from jax.experimental.pallas import tpu_sc as plsc

_B, _S = 4096, 50
_D = 32
_NC, _NS = 2, 16
_NW = _NC * _NS
_BB = _B // _NW


@jax.jit
def _embed(t128, xt):
    mesh = plsc.VectorSubcoreMesh(core_axis_name="c", subcore_axis_name="s")

    @functools.partial(
        pl.kernel,
        mesh=mesh,
        out_type=jax.ShapeDtypeStruct((_S, _D, _B), jnp.float32),
        scratch_types=[
            pltpu.VMEM((_S, _BB), jnp.int32),
            pltpu.VMEM((_S, _BB), jnp.int32),
            pltpu.VMEM((_BB, 128), jnp.float32),
            pltpu.VMEM((_BB, 128), jnp.float32),
            pltpu.VMEM((_D, _BB), jnp.float32),
            pltpu.VMEM((_D, _BB), jnp.float32),
            pltpu.SemaphoreType.DMA,
            pltpu.SemaphoreType.DMA,
            pltpu.SemaphoreType.DMA,
            pltpu.SemaphoreType.DMA,
        ],
        compiler_params=pltpu.CompilerParams(
            use_tc_tiling_on_sc=True, needs_layout_passes=False
        ),
    )
    def k(tbl, xt_hbm, out, idx_v, jv, g0, g1, t0, t1, gs0, gs1, ss0, ss1):
        w = lax.axis_index("s") * _NC + lax.axis_index("c")
        pltpu.sync_copy(xt_hbm.at[:, pl.ds(w * _BB, _BB)], idx_v)

        gbuf, tbuf = (g0, g1), (t0, t1)
        gsem, ssem = (gs0, gs1), (ss0, ss1)
        lanes = lax.iota(jnp.int32, 16)
        rowvec = [lanes + 16 * p for p in range(8)]

        def prep(s, carry):
            for p in range(8):
                iv = idx_v[s, pl.ds(16 * p, 16)]
                jv[s, pl.ds(16 * p, 16)] = lax.shift_right_logical(iv, 2)
            return carry

        lax.fori_loop(0, _S, prep, 0)

        def fire(s, par):
            pltpu.async_copy(tbl.at[jv.at[s]], gbuf[par], gsem[par])

        def drain_gather(par):
            pltpu.make_async_copy(
                tbl.at[pl.ds(0, _BB)], gbuf[par], gsem[par]
            ).wait()

        def drain_store(par):
            pltpu.make_async_copy(
                out.at[0, :, pl.ds(0, _BB)], tbuf[par], ssem[par]
            ).wait()

        def transpose_select(s, par):
            g, t = gbuf[par], tbuf[par]
            q32 = []
            for p in range(8):
                iv = idx_v[s, pl.ds(16 * p, 16)]
                q32.append(lax.shift_left(jnp.bitwise_and(iv, 3), 5))
            for m0 in range(0, 2 * _BB, 8):
                vals = []
                for m in range(m0, m0 + 8):
                    d, p = m // 8, m % 8
                    vals.append(plsc.load_gather(g, [rowvec[p], q32[p] + d]))
                for m in range(m0, m0 + 8):
                    d, p = m // 8, m % 8
                    t[d, pl.ds(16 * p, 16)] = vals[m - m0]

        def handle(i, s, par):
            drain_gather(par)

            @pl.when(s + 1 < _S)
            def _():
                fire(s + 1, 1 - par)

            @pl.when(i > 0)
            def _():
                drain_store(par)

            transpose_select(s, par)
            pltpu.async_copy(
                tbuf[par], out.at[s, :, pl.ds(w * _BB, _BB)], ssem[par]
            )

        fire(0, 0)

        def body(i, carry):
            handle(i, 2 * i, 0)
            handle(i, 2 * i + 1, 1)
            return carry

        lax.fori_loop(0, _S // 2, body, 0)
        drain_store(0)
        drain_store(1)

    return k(t128, xt)


def kernel(x, table):
    t128 = table.reshape(_D * 1000000 // 128, 128)
    xt = jnp.transpose(x)
    out = _embed(t128, xt)
    return jnp.transpose(out, (2, 0, 1))
